# Initial kernel scaffold; baseline (speedup 1.0000x reference)
import jax
import jax.numpy as jnp
from jax.experimental import pallas as pl


def kernel(queries, keys, values, attn_mask, rel_pos_bias_table):
    raise NotImplementedError("write your pallas kernel here")


# speedup vs baseline: 38.4942x; 38.4942x over previous
import math

import numpy as np
import jax
import jax.numpy as jnp
from jax.experimental import pallas as pl
from jax.experimental.pallas import tpu as pltpu

_WS = 46
_FACTOR = 2
_NEG_INF = float("-inf")


def _rpi_slice(n_rows, n_cols):
    ws = _WS
    idx = np.arange(ws * ws)
    r, c = idx // ws, idx % ws
    ar, ac = r[:n_rows, None], c[:n_rows, None]
    br, bc = r[None, :n_cols], c[None, :n_cols]
    return ((ar - br + ws - 1) * (2 * ws - 1) + (ac - bc + ws - 1)).astype(np.int32)


def _body(q_ref, k_ref, kf_ref, v_ref, bias_ref, ctx_ref, attn_ref, *, u, q_chunk, c_chunk):
    L, D = q_ref.shape[2], q_ref.shape[3]
    scale = 1.0 / math.sqrt(D)

    Q = q_ref[0, 0]
    K = k_ref[0, 0]
    V = v_ref[0, 0]
    Ks = kf_ref[0, 0][:, :D]

    m_parts = []
    for i in range(L // q_chunk):
        qc = Q[i * q_chunk:(i + 1) * q_chunk]
        s = jax.lax.dot_general(qc, Ks, (((1,), (1,)), ((), ())),
                                preferred_element_type=jnp.float32)
        m_parts.append(jnp.max(s, axis=1, keepdims=True)
                       - jnp.sum(s, axis=1, keepdims=True) * (1.0 / L))
    M = jnp.concatenate(m_parts, axis=0)

    iota = jax.lax.broadcasted_iota(jnp.int32, (L, 1), 0)
    idxs = []
    Mw = M
    for _ in range(u):
        mval = jnp.max(Mw)
        idx = jnp.min(jnp.where(Mw >= mval, iota, L))
        idxs.append(idx)
        Mw = jnp.where(iota == idx, _NEG_INF, Mw)

    Qr = jnp.concatenate([q_ref[0, 0, pl.ds(idxs[i], 1), :] for i in range(u)],
                         axis=0)

    scores = jax.lax.dot_general(Qr, K, (((1,), (1,)), ((), ())),
                                 preferred_element_type=jnp.float32)
    scores = (scores + bias_ref[...]) * scale
    smax = jnp.max(scores, axis=1, keepdims=True)
    e = jnp.exp(scores - smax)
    attn = e / jnp.sum(e, axis=1, keepdims=True)
    attn_ref[0, 0] = attn
    update = jnp.dot(attn, V, preferred_element_type=jnp.float32)

    tri = (jax.lax.broadcasted_iota(jnp.int32, (c_chunk, c_chunk), 0)
           >= jax.lax.broadcasted_iota(jnp.int32, (c_chunk, c_chunk), 1)
           ).astype(jnp.float32)
    carry = jnp.zeros((1, D), jnp.float32)
    for i in range(L // c_chunk):
        vc = V[i * c_chunk:(i + 1) * c_chunk]
        pc = jnp.dot(tri, vc, preferred_element_type=jnp.float32,
                     precision=jax.lax.Precision.HIGHEST) + carry
        ctx_ref[0, 0, i * c_chunk:(i + 1) * c_chunk, :] = pc
        carry = pc[c_chunk - 1:c_chunk, :]

    for i in range(u):
        ctx_ref[0, 0, pl.ds(idxs[i], 1), :] = update[i:i + 1, :]


def kernel(queries, keys, values, attn_mask, rel_pos_bias_table):
    del attn_mask
    B, L, H, D = queries.shape
    u = min(_FACTOR * int(np.ceil(np.log(L))), L)

    Qt = jnp.transpose(queries, (0, 2, 1, 3))
    Kt = jnp.transpose(keys, (0, 2, 1, 3))
    Kf = Kt.reshape(B, H, L // 2, 2 * D)
    Vt = jnp.transpose(values, (0, 2, 1, 3))
    bias = rel_pos_bias_table[jnp.asarray(_rpi_slice(u, L)), 0]

    import functools
    body = functools.partial(_body, u=u, q_chunk=min(512, L), c_chunk=min(512, L))
    ctx, attn = pl.pallas_call(
        body,
        grid=(B, H),
        in_specs=[
            pl.BlockSpec((1, 1, L, D), lambda b, h: (b, h, 0, 0)),
            pl.BlockSpec((1, 1, L, D), lambda b, h: (b, h, 0, 0)),
            pl.BlockSpec((1, 1, L // 2, 2 * D), lambda b, h: (b, h, 0, 0)),
            pl.BlockSpec((1, 1, L, D), lambda b, h: (b, h, 0, 0)),
            pl.BlockSpec((u, L), lambda b, h: (0, 0)),
        ],
        out_specs=[
            pl.BlockSpec((1, 1, L, D), lambda b, h: (b, h, 0, 0)),
            pl.BlockSpec((1, 1, u, L), lambda b, h: (b, h, 0, 0)),
        ],
        out_shape=[
            jax.ShapeDtypeStruct((B, H, L, D), jnp.float32),
            jax.ShapeDtypeStruct((B, H, u, L), jnp.float32),
        ],
        compiler_params=pltpu.CompilerParams(
            dimension_semantics=("parallel", "parallel")),
    )(Qt, Kt, Kf, Vt, bias)
    return jnp.transpose(ctx, (0, 2, 1, 3)), attn


# speedup vs baseline: 57.7454x; 1.5001x over previous
import math

import numpy as np
import jax
import jax.numpy as jnp
from jax.experimental import pallas as pl
from jax.experimental.pallas import tpu as pltpu

_WS = 46
_FACTOR = 2
_NEG_INF = float("-inf")


def _rpi_slice(n_rows, n_cols):
    ws = _WS
    idx = np.arange(ws * ws)
    r, c = idx // ws, idx % ws
    ar, ac = r[:n_rows, None], c[:n_rows, None]
    br, bc = r[None, :n_cols], c[None, :n_cols]
    return ((ar - br + ws - 1) * (2 * ws - 1) + (ac - bc + ws - 1)).astype(np.int32)


def _body(q_ref, k_ref, kf_ref, v_ref, bias_ref, ctx_ref, attn_ref, *, u, q_chunk, c_chunk):
    L, D = q_ref.shape[2], q_ref.shape[3]
    scale = 1.0 / math.sqrt(D)

    Q = q_ref[0, 0]
    K = k_ref[0, 0]
    V = v_ref[0, 0]
    Ks = kf_ref[0, 0][:, :D]

    m_parts = []
    for i in range(L // q_chunk):
        qc = Q[i * q_chunk:(i + 1) * q_chunk]
        s = jax.lax.dot_general(qc, Ks, (((1,), (1,)), ((), ())),
                                preferred_element_type=jnp.float32)
        m_parts.append(jnp.max(s, axis=1, keepdims=True)
                       - jnp.sum(s, axis=1, keepdims=True) * (1.0 / L))
    M = jnp.transpose(jnp.concatenate(m_parts, axis=0))

    iota = jax.lax.broadcasted_iota(jnp.int32, (1, L), 1)
    idxs = []
    Mw = M
    for _ in range(u):
        mval = jnp.max(Mw)
        idx = jnp.min(jnp.where(Mw >= mval, iota, L))
        idxs.append(idx)
        Mw = jnp.where(iota == idx, _NEG_INF, Mw)

    Qr = jnp.concatenate([q_ref[0, 0, pl.ds(idxs[i], 1), :] for i in range(u)],
                         axis=0)

    scores = jax.lax.dot_general(Qr, K, (((1,), (1,)), ((), ())),
                                 preferred_element_type=jnp.float32)
    scores = (scores + bias_ref[...]) * scale
    smax = jnp.max(scores, axis=1, keepdims=True)
    e = jnp.exp(scores - smax)
    attn = e / jnp.sum(e, axis=1, keepdims=True)
    attn_ref[0, 0] = attn
    update = jnp.dot(attn, V, preferred_element_type=jnp.float32)

    tri = (jax.lax.broadcasted_iota(jnp.int32, (c_chunk, c_chunk), 0)
           >= jax.lax.broadcasted_iota(jnp.int32, (c_chunk, c_chunk), 1)
           ).astype(jnp.float32)
    carry = jnp.zeros((1, D), jnp.float32)
    for i in range(L // c_chunk):
        vc = V[i * c_chunk:(i + 1) * c_chunk]
        pc = jnp.dot(tri, vc, preferred_element_type=jnp.float32,
                     precision=jax.lax.Precision.HIGHEST) + carry
        ctx_ref[0, 0, i * c_chunk:(i + 1) * c_chunk, :] = pc
        carry = pc[c_chunk - 1:c_chunk, :]

    for i in range(u):
        ctx_ref[0, 0, pl.ds(idxs[i], 1), :] = update[i:i + 1, :]


def kernel(queries, keys, values, attn_mask, rel_pos_bias_table):
    del attn_mask
    B, L, H, D = queries.shape
    u = min(_FACTOR * int(np.ceil(np.log(L))), L)

    Qt = jnp.transpose(queries, (0, 2, 1, 3))
    Kt = jnp.transpose(keys, (0, 2, 1, 3))
    Kf = Kt.reshape(B, H, L // 2, 2 * D)
    Vt = jnp.transpose(values, (0, 2, 1, 3))
    bias = rel_pos_bias_table[jnp.asarray(_rpi_slice(u, L)), 0]

    import functools
    body = functools.partial(_body, u=u, q_chunk=min(512, L), c_chunk=min(128, L))
    ctx, attn = pl.pallas_call(
        body,
        grid=(B, H),
        in_specs=[
            pl.BlockSpec((1, 1, L, D), lambda b, h: (b, h, 0, 0)),
            pl.BlockSpec((1, 1, L, D), lambda b, h: (b, h, 0, 0)),
            pl.BlockSpec((1, 1, L // 2, 2 * D), lambda b, h: (b, h, 0, 0)),
            pl.BlockSpec((1, 1, L, D), lambda b, h: (b, h, 0, 0)),
            pl.BlockSpec((u, L), lambda b, h: (0, 0)),
        ],
        out_specs=[
            pl.BlockSpec((1, 1, L, D), lambda b, h: (b, h, 0, 0)),
            pl.BlockSpec((1, 1, u, L), lambda b, h: (b, h, 0, 0)),
        ],
        out_shape=[
            jax.ShapeDtypeStruct((B, H, L, D), jnp.float32),
            jax.ShapeDtypeStruct((B, H, u, L), jnp.float32),
        ],
        compiler_params=pltpu.CompilerParams(
            dimension_semantics=("parallel", "parallel")),
    )(Qt, Kt, Kf, Vt, bias)
    return jnp.transpose(ctx, (0, 2, 1, 3)), attn
